# baseline (speedup 1.0000x reference)
import jax
import jax.numpy as jnp
from jax.experimental import pallas as pl

_N = 8192
_K = 819
_R = 64
_C = 128


def _body(g_ref, o_ref):
    g = g_ref[...]
    u = jax.lax.bitcast_convert_type(g, jnp.uint32)
    sign = u >> jnp.uint32(31)
    flip = jnp.uint32(0x80000000) + sign * jnp.uint32(0x7FFFFFFF)
    key = u ^ flip

    kk = jnp.int32(_K)

    tstar = jnp.uint32(0)
    rounds = [(27 - 5 * r, 31) for r in range(6)] + [(0, 3)]
    for shift, nc in rounds:
        cnts = [
            jnp.sum((key >= (tstar + jnp.uint32(i << shift)))
                    .astype(jnp.int32))
            for i in range(1, nc + 1)
        ]
        m = jnp.uint32(0)
        for c in cnts:
            m = m + (c >= kk).astype(jnp.uint32)
        tstar = tstar + (m << jnp.uint32(shift))

    gt = key > tstar
    eq = key == tstar
    c = jnp.sum(gt.astype(jnp.int32))
    need = (kk - c).astype(jnp.float32)

    e = eq.astype(jnp.float32)
    i1 = jax.lax.broadcasted_iota(jnp.int32, (_C, _C), 0)
    j1 = jax.lax.broadcasted_iota(jnp.int32, (_C, _C), 1)
    upper = (i1 <= j1).astype(jnp.float32)
    rowcum = jnp.dot(e, upper, preferred_element_type=jnp.float32)

    i2 = jax.lax.broadcasted_iota(jnp.int32, (_R, _R), 0)
    j2 = jax.lax.broadcasted_iota(jnp.int32, (_R, _R), 1)
    strict_lower = (j2 < i2).astype(jnp.float32)
    colpref = jnp.dot(strict_lower, e, preferred_element_type=jnp.float32)
    row_prefix = jnp.sum(colpref, axis=1, keepdims=True)

    rank = rowcum + row_prefix
    sel = jnp.logical_and(eq, rank <= need)
    o_ref[...] = jnp.logical_or(gt, sel).astype(jnp.float32)


def kernel(x, gate_scores):
    g2 = gate_scores.reshape(_R, _C)
    mask = pl.pallas_call(
        _body,
        out_shape=jax.ShapeDtypeStruct((_R, _C), jnp.float32),
    )(g2)
    return mask.reshape(_N).astype(x.dtype)


# speedup vs baseline: 1.0481x; 1.0481x over previous
import jax
import jax.numpy as jnp
from jax.experimental import pallas as pl

_N = 8192
_K = 819
_R = 64
_C = 128


def _body(g_ref, o_ref):
    g = g_ref[...]
    u = jax.lax.bitcast_convert_type(g, jnp.uint32)
    sign = u >> jnp.uint32(31)
    flip = jnp.uint32(0x80000000) + sign * jnp.uint32(0x7FFFFFFF)
    key = u ^ flip

    kk = jnp.int32(_K)

    tstar = jnp.uint32(0)
    rounds = [(28 - 4 * r, 15) for r in range(8)]
    for shift, nc in rounds:
        cnts = [
            jnp.sum((key >= (tstar + jnp.uint32(i << shift)))
                    .astype(jnp.int32))
            for i in range(1, nc + 1)
        ]
        m = jnp.uint32(0)
        for c in cnts:
            m = m + (c >= kk).astype(jnp.uint32)
        tstar = tstar + (m << jnp.uint32(shift))

    gt = key > tstar
    eq = key == tstar
    c = jnp.sum(gt.astype(jnp.int32))
    need = (kk - c).astype(jnp.float32)

    e = eq.astype(jnp.float32)
    i1 = jax.lax.broadcasted_iota(jnp.int32, (_C, _C), 0)
    j1 = jax.lax.broadcasted_iota(jnp.int32, (_C, _C), 1)
    upper = (i1 <= j1).astype(jnp.float32)
    rowcum = jnp.dot(e, upper, preferred_element_type=jnp.float32)

    i2 = jax.lax.broadcasted_iota(jnp.int32, (_R, _R), 0)
    j2 = jax.lax.broadcasted_iota(jnp.int32, (_R, _R), 1)
    strict_lower = (j2 < i2).astype(jnp.float32)
    colpref = jnp.dot(strict_lower, e, preferred_element_type=jnp.float32)
    row_prefix = jnp.sum(colpref, axis=1, keepdims=True)

    rank = rowcum + row_prefix
    sel = jnp.logical_and(eq, rank <= need)
    o_ref[...] = jnp.logical_or(gt, sel).astype(jnp.float32)


def kernel(x, gate_scores):
    g2 = gate_scores.reshape(_R, _C)
    mask = pl.pallas_call(
        _body,
        out_shape=jax.ShapeDtypeStruct((_R, _C), jnp.float32),
    )(g2)
    return mask.reshape(_N).astype(x.dtype)
